# baseline (speedup 1.0000x reference)
import jax
import jax.numpy as jnp
from jax import lax
from jax.experimental import pallas as pl
from jax.experimental.pallas import tpu as pltpu
from jax.experimental.pallas import tpu_sc as plsc

NUM_CORES = 1
NUM_SUBCORES = 16
NUM_WORKERS = NUM_CORES * NUM_SUBCORES
LANES = 16
BATCH = 16384
EMB = 64
NPAIR = 500000
BPW = BATCH // NUM_WORKERS
WAVE = 128
NWAVES = BPW // WAVE
NCHUNK = EMB // LANES

_GATHER_DNUMS = lax.GatherDimensionNumbers(
    offset_dims=(), collapsed_slice_dims=(0,), start_index_map=(0,))


def _shuffle(x, perm):
    return lax.gather(x, perm[:, None], dimension_numbers=_GATHER_DNUMS,
                      slice_sizes=(1,),
                      mode=lax.GatherScatterMode.PROMISE_IN_BOUNDS)


def _body(u_hbm, v_hbm, ue_hbm, ve_hbm, out_hbm,
          u_idx, v_idx, ublk, vblk, ue_w, ve_w, out_v, sem0, sem1):
    wid = lax.axis_index("s") * NUM_CORES + lax.axis_index("c")
    base = wid * BPW

    for j in range(NWAVES):
        pltpu.sync_copy(u_hbm.at[pl.ds(base + j * WAVE, WAVE)],
                        u_idx.at[pl.ds(j * WAVE, WAVE)])
        pltpu.sync_copy(v_hbm.at[pl.ds(base + j * WAVE, WAVE)],
                        v_idx.at[pl.ds(j * WAVE, WAVE)])

    for j in range(NWAVES):
        for k in range(WAVE // LANES):
            off = j * WAVE + k * LANES
            ublk[j, pl.ds(k * LANES, LANES)] = u_idx[pl.ds(off, LANES)] >> 1
            vblk[j, pl.ds(k * LANES, LANES)] = v_idx[pl.ds(off, LANES)] >> 1

    sems = (sem0, sem1)

    def fire(j):
        pltpu.async_copy(ue_hbm.at[ublk.at[j]], ue_w.at[j % 2], sems[j % 2])
        pltpu.async_copy(ve_hbm.at[vblk.at[j]], ve_w.at[j % 2], sems[j % 2])

    def drain(j):
        pltpu.make_async_copy(ue_hbm.at[ublk.at[j]], ue_w.at[j % 2], sems[j % 2]).wait()
        pltpu.make_async_copy(ve_hbm.at[vblk.at[j]], ve_w.at[j % 2], sems[j % 2]).wait()

    lanes = lax.iota(jnp.int32, LANES)
    lanes_f = lanes.astype(jnp.float32)
    perms = [lanes ^ (1 << t) for t in range(4)]
    one = jnp.ones((LANES,), jnp.float32)
    onehots = [jnp.maximum(one - jnp.abs(lanes_f - float(r)), 0.0)
               for r in range(LANES)]

    fire(0)
    for j in range(NWAVES):
        if j + 1 < NWAVES:
            fire(j + 1)
        drain(j)
        uw, vw = ue_w.at[j % 2], ve_w.at[j % 2]

        def group(g, carry, j=j, uw=uw, vw=vw):
            gbase = pl.multiple_of(g * LANES, LANES)
            uraw = u_idx[pl.ds(j * WAVE + gbase, LANES)]
            vraw = v_idx[pl.ds(j * WAVE + gbase, LANES)]
            uh = (uraw & 1).astype(jnp.float32)
            vh = (vraw & 1).astype(jnp.float32)
            sums = jnp.zeros((LANES,), jnp.float32)
            for r in range(LANES):
                slot = gbase + r
                sel_r = jnp.full((LANES,), r, jnp.int32)
                hu = _shuffle(uh, sel_r)
                hv = _shuffle(vh, sel_r)
                acc = jnp.zeros((LANES,), jnp.float32)
                for c in range(NCHUNK):
                    ulo = uw[slot, pl.ds(c * LANES, LANES)]
                    uhi = uw[slot, pl.ds(EMB + c * LANES, LANES)]
                    vlo = vw[slot, pl.ds(c * LANES, LANES)]
                    vhi = vw[slot, pl.ds(EMB + c * LANES, LANES)]
                    ue = ulo + (uhi - ulo) * hu
                    ve = vlo + (vhi - vlo) * hv
                    acc = acc + ue * ve
                for t in range(4):
                    acc = acc + _shuffle(acc, perms[t])
                sums = sums + acc * onehots[r]
            out_v[pl.ds(j * WAVE + gbase, LANES)] = sums
            return carry

        lax.fori_loop(0, WAVE // LANES, group, 0)

    pltpu.sync_copy(out_v, out_hbm.at[pl.ds(base, BPW)])


@jax.jit
def kernel(u, v, user_emb, item_emb):
    mesh = plsc.VectorSubcoreMesh(core_axis_name="c", subcore_axis_name="s",
                                  num_cores=NUM_CORES, num_subcores=NUM_SUBCORES)
    run = pl.kernel(
        _body,
        out_type=jax.ShapeDtypeStruct((BATCH,), jnp.float32),
        mesh=mesh,
        scratch_types=[
            pltpu.VMEM((BPW,), jnp.int32),
            pltpu.VMEM((BPW,), jnp.int32),
            pltpu.VMEM((NWAVES, WAVE), jnp.int32),
            pltpu.VMEM((NWAVES, WAVE), jnp.int32),
            pltpu.VMEM((2, WAVE, 2 * EMB), jnp.float32),
            pltpu.VMEM((2, WAVE, 2 * EMB), jnp.float32),
            pltpu.VMEM((BPW,), jnp.float32),
            pltpu.SemaphoreType.DMA,
            pltpu.SemaphoreType.DMA,
        ],
        compiler_params=pltpu.CompilerParams(use_tc_tiling_on_sc=True),
    )
    ue_pair = user_emb.reshape(NPAIR, 2 * EMB)
    ve_pair = item_emb.reshape(NPAIR, 2 * EMB)
    return run(u, v, ue_pair, ve_pair)


# speedup vs baseline: 1.0273x; 1.0273x over previous
import jax
import jax.numpy as jnp
from jax import lax
from jax.experimental import pallas as pl
from jax.experimental.pallas import tpu as pltpu
from jax.experimental.pallas import tpu_sc as plsc

NUM_CORES = 2
NUM_SUBCORES = 16
NUM_WORKERS = NUM_CORES * NUM_SUBCORES
LANES = 16
BATCH = 16384
EMB = 64
BPW = BATCH // NUM_WORKERS
CHUNK = 128
NCHUNK = BPW // CHUNK

_GATHER_DNUMS = lax.GatherDimensionNumbers(
    offset_dims=(), collapsed_slice_dims=(0,), start_index_map=(0,))


def _shuffle(x, perm):
    return lax.gather(x, perm[:, None], dimension_numbers=_GATHER_DNUMS,
                      slice_sizes=(1,),
                      mode=lax.GatherScatterMode.PROMISE_IN_BOUNDS)


def _body(u_hbm, v_hbm, ue_hbm, ve_hbm, out_hbm,
          u_idx, v_idx, ue_v, ve_v, out_v, sem):
    wid = lax.axis_index("s") * NUM_CORES + lax.axis_index("c")
    base = wid * BPW

    for j in range(NCHUNK):
        pltpu.sync_copy(u_hbm.at[pl.ds(base + j * CHUNK, CHUNK)], u_idx.at[j])
        pltpu.sync_copy(v_hbm.at[pl.ds(base + j * CHUNK, CHUNK)], v_idx.at[j])

    for j in range(NCHUNK):
        pltpu.async_copy(ue_hbm.at[u_idx.at[j]], ue_v.at[pl.ds(j * CHUNK, CHUNK)], sem)
        pltpu.async_copy(ve_hbm.at[v_idx.at[j]], ve_v.at[pl.ds(j * CHUNK, CHUNK)], sem)
    for j in range(NCHUNK):
        pltpu.make_async_copy(ue_hbm.at[u_idx.at[j]], ue_v.at[pl.ds(j * CHUNK, CHUNK)], sem).wait()
        pltpu.make_async_copy(ve_hbm.at[v_idx.at[j]], ve_v.at[pl.ds(j * CHUNK, CHUNK)], sem).wait()

    lanes = lax.iota(jnp.int32, LANES)
    perms = [lanes ^ (1 << t) for t in range(4)]

    def group(g, carry):
        gbase = pl.multiple_of(g * LANES, LANES)
        sums = jnp.zeros((LANES,), jnp.float32)
        for r in range(LANES):
            row = gbase + r
            acc = ue_v[row, pl.ds(0, LANES)] * ve_v[row, pl.ds(0, LANES)]
            for c in range(1, EMB // LANES):
                acc = acc + (ue_v[row, pl.ds(c * LANES, LANES)]
                             * ve_v[row, pl.ds(c * LANES, LANES)])
            for t in range(4):
                acc = acc + _shuffle(acc, perms[t])
            sums = jnp.where(lanes == r, acc, sums)
        out_v[pl.ds(gbase, LANES)] = sums
        return carry

    lax.fori_loop(0, BPW // LANES, group, 0)

    pltpu.sync_copy(out_v, out_hbm.at[pl.ds(base, BPW)])


@jax.jit
def kernel(u, v, user_emb, item_emb):
    mesh = plsc.VectorSubcoreMesh(core_axis_name="c", subcore_axis_name="s",
                                  num_cores=NUM_CORES, num_subcores=NUM_SUBCORES)
    run = pl.kernel(
        _body,
        out_type=jax.ShapeDtypeStruct((BATCH,), jnp.float32),
        mesh=mesh,
        scratch_types=[
            pltpu.VMEM((NCHUNK, CHUNK), jnp.int32),
            pltpu.VMEM((NCHUNK, CHUNK), jnp.int32),
            pltpu.VMEM((BPW, EMB), jnp.float32),
            pltpu.VMEM((BPW, EMB), jnp.float32),
            pltpu.VMEM((BPW,), jnp.float32),
            pltpu.SemaphoreType.DMA,
        ],
        compiler_params=pltpu.CompilerParams(use_tc_tiling_on_sc=False),
    )
    return run(u, v, user_emb, item_emb)
